# baseline (speedup 1.0000x reference)
import jax
import jax.numpy as jnp
from jax.experimental import pallas as pl
from jax.experimental.pallas import tpu as pltpu

_B = 128
_V = 100000
_BR = 16
_NCH = _B // _BR


def _chunk_copy(x_hbm, buf, sem, j):
    return pltpu.make_async_copy(
        x_hbm.at[pl.ds(j * _BR, _BR), :],
        buf,
        sem.at[j],
    )


def _lse_pick_kernel(a_smem, a_vmem, x_hbm, o_ref, pick8, *bufs_and_sems):
    bufs = bufs_and_sems[:_NCH]
    psem = bufs_and_sems[_NCH]
    sem = bufs_and_sems[_NCH + 1]

    pick_copies = []
    for b in range(_B):
        start = pl.multiple_of((a_smem[b, 0] // 128) * 128, 128)
        cp = pltpu.make_async_copy(
            x_hbm.at[pl.ds((b // 8) * 8, 8), pl.ds(start, 128)],
            pick8.at[b],
            psem,
        )
        cp.start()
        pick_copies.append(cp)

    for j in range(_NCH):
        _chunk_copy(x_hbm, bufs[j], sem, j).start()

    for cp in pick_copies:
        cp.wait()
    a = a_vmem[...]
    rem = (a - (a // 128) * 128).reshape(_B, 1, 1)
    i0 = jax.lax.broadcasted_iota(jnp.int32, (_B, 8, 128), 0)
    i1 = jax.lax.broadcasted_iota(jnp.int32, (_B, 8, 128), 1)
    i2 = jax.lax.broadcasted_iota(jnp.int32, (_B, 8, 128), 2)
    hit = (i1 == i0 - (i0 // 8) * 8) & (i2 == rem)
    pick = jnp.sum(jnp.where(hit, pick8[...], 0.0), axis=(1, 2)).reshape(_B, 1)

    for j in range(_NCH):
        _chunk_copy(x_hbm, bufs[j], sem, j).wait()
        x = bufs[j][...]
        m = jnp.max(x, axis=-1, keepdims=True)
        s = jnp.sum(jnp.exp(x - m), axis=-1, keepdims=True)
        o_ref[pl.ds(j * _BR, _BR), :] = pick[j * _BR : (j + 1) * _BR, :] - (
            m + jnp.log(s)
        )


@jax.jit
def kernel(logits, actions):
    out = pl.pallas_call(
        _lse_pick_kernel,
        in_specs=[
            pl.BlockSpec(memory_space=pltpu.MemorySpace.SMEM),
            pl.BlockSpec(memory_space=pltpu.MemorySpace.VMEM),
            pl.BlockSpec(memory_space=pltpu.MemorySpace.HBM),
        ],
        out_specs=pl.BlockSpec(memory_space=pltpu.MemorySpace.VMEM),
        out_shape=jax.ShapeDtypeStruct((_B, 1), jnp.float32),
        scratch_shapes=[pltpu.VMEM((_B, 8, 128), jnp.float32)]
        + [pltpu.VMEM((_BR, _V), jnp.float32) for _ in range(_NCH)]
        + [pltpu.SemaphoreType.DMA, pltpu.SemaphoreType.DMA((_NCH,))],
    )(actions, actions, logits)
    return out


# speedup vs baseline: 1.1371x; 1.1371x over previous
import jax
import jax.numpy as jnp
from jax.experimental import pallas as pl
from jax.experimental.pallas import tpu as pltpu

_B = 128
_V = 100000
_BR = 8
_NBLK = _B // _BR
_NS = 5
_SV = 20096


def _lse_pick_kernel(a_ref, *refs):
    x_refs = refs[:_NS]
    o_ref = refs[_NS]
    a = a_ref[...]

    base = jax.lax.broadcasted_iota(jnp.int32, (_BR, _SV), 1)
    xs = [r[...] for r in x_refs]
    xs[-1] = jnp.where(base < _V - (_NS - 1) * _SV, xs[-1], -jnp.inf)

    m = xs[0].max(axis=-1, keepdims=True)
    for x in xs[1:]:
        m = jnp.maximum(m, x.max(axis=-1, keepdims=True))

    s = jnp.zeros((_BR, 1), jnp.float32)
    pick = jnp.zeros((_BR, 1), jnp.float32)
    for i, x in enumerate(xs):
        s = s + jnp.sum(jnp.exp(x - m), axis=-1, keepdims=True)
        hit = base == a - i * _SV
        pick = pick + jnp.sum(jnp.where(hit, x, 0.0), axis=-1, keepdims=True)

    o_ref[...] = pick - (m + jnp.log(s))


@jax.jit
def kernel(logits, actions):
    out = pl.pallas_call(
        _lse_pick_kernel,
        grid=(_NBLK,),
        in_specs=[pl.BlockSpec((_BR, 1), lambda j: (j, 0))]
        + [
            pl.BlockSpec((_BR, _SV), lambda j, i=i: (j, i))
            for i in range(_NS)
        ],
        out_specs=pl.BlockSpec((_BR, 1), lambda j: (j, 0)),
        out_shape=jax.ShapeDtypeStruct((_B, 1), jnp.float32),
        compiler_params=pltpu.CompilerParams(
            dimension_semantics=("arbitrary",),
        ),
    )(actions, *([logits] * _NS))
    return out
